# Initial kernel scaffold; baseline (speedup 1.0000x reference)
import jax
import jax.numpy as jnp
from jax.experimental import pallas as pl


def kernel(input_ids, table):
    raise NotImplementedError("write your pallas kernel here")


# speedup vs baseline: 1.6390x; 1.6390x over previous
import functools

import jax
import jax.numpy as jnp
from jax import lax
from jax.experimental import pallas as pl
from jax.experimental.pallas import tpu as pltpu
from jax.experimental.pallas import tpu_sc as plsc

EMBED_DIM = 128
NUM_CORES = 2
NUM_SUBCORES = 16
NW = NUM_CORES * NUM_SUBCORES

CHUNK = 512
SUB = 128


def _embed_gather(table, ids):
    B = ids.shape[0]
    rows_per_w = B // NW
    nchunk = rows_per_w // CHUNK
    subs_per_chunk = CHUNK // SUB

    mesh = plsc.VectorSubcoreMesh(core_axis_name="c", subcore_axis_name="s")

    @functools.partial(
        pl.kernel,
        mesh=mesh,
        out_type=jax.ShapeDtypeStruct((B, EMBED_DIM), jnp.float32),
        scratch_types=[
            pltpu.VMEM((CHUNK,), jnp.int32),
            pltpu.VMEM((CHUNK, EMBED_DIM), jnp.float32),
            pltpu.SemaphoreType.DMA,
        ],
    )
    def k(table_hbm, ids_hbm, out_hbm, idx_v, rows_v, sem):
        wid = lax.axis_index("s") * NUM_CORES + lax.axis_index("c")
        base = wid * rows_per_w

        def body(c, carry):
            off = base + c * CHUNK
            pltpu.sync_copy(ids_hbm.at[pl.ds(off, CHUNK)], idx_v)
            cps = []
            for j in range(subs_per_chunk):
                cps.append(
                    pltpu.async_copy(
                        table_hbm.at[idx_v.at[pl.ds(j * SUB, SUB)]],
                        rows_v.at[pl.ds(j * SUB, SUB)],
                        sem,
                    )
                )
            for cp in cps:
                cp.wait()
            pltpu.sync_copy(rows_v, out_hbm.at[pl.ds(off, CHUNK)])
            return carry

        lax.fori_loop(0, nchunk, body, 0)

    return k(table, ids)


def kernel(input_ids, table):
    batch, seq = input_ids.shape
    ids = input_ids.reshape(-1)
    out = _embed_gather(table, ids)
    return out.reshape(batch, seq, EMBED_DIM)


# speedup vs baseline: 1.6727x; 1.0206x over previous
import functools

import jax
import jax.numpy as jnp
from jax import lax
from jax.experimental import pallas as pl
from jax.experimental.pallas import tpu as pltpu
from jax.experimental.pallas import tpu_sc as plsc

EMBED_DIM = 128
NUM_CORES = 2
NUM_SUBCORES = 16
NW = NUM_CORES * NUM_SUBCORES

CHUNK = 256
SUB = 128
NBUF = 2


def _embed_gather(table, ids):
    B = ids.shape[0]
    rows_per_w = B // NW
    nchunk = rows_per_w // CHUNK
    subs_per_chunk = CHUNK // SUB

    mesh = plsc.VectorSubcoreMesh(core_axis_name="c", subcore_axis_name="s")

    @functools.partial(
        pl.kernel,
        mesh=mesh,
        out_type=jax.ShapeDtypeStruct((B, EMBED_DIM), jnp.float32),
        scratch_types=[
            pltpu.VMEM((NBUF, CHUNK), jnp.int32),
            pltpu.VMEM((NBUF, CHUNK, EMBED_DIM), jnp.float32),
            pltpu.SemaphoreType.DMA,
            pltpu.SemaphoreType.DMA,
            pltpu.SemaphoreType.DMA,
            pltpu.SemaphoreType.DMA,
        ],
    )
    def k(table_hbm, ids_hbm, out_hbm, idx_v, rows_v, g0, g1, s0, s1):
        gsem = [g0, g1]
        ssem = [s0, s1]
        wid = lax.axis_index("s") * NUM_CORES + lax.axis_index("c")
        base = wid * rows_per_w

        def body(g, carry):
            for b in range(NBUF):
                c = NBUF * g + b
                off = base + c * CHUNK

                @pl.when(c >= NBUF)
                def _(b=b, off=off):
                    pltpu.make_async_copy(
                        rows_v.at[b],
                        out_hbm.at[pl.ds(off - NBUF * CHUNK, CHUNK)],
                        ssem[b],
                    ).wait()

                pltpu.sync_copy(ids_hbm.at[pl.ds(off, CHUNK)], idx_v.at[b])
                cps = []
                for j in range(subs_per_chunk):
                    cps.append(
                        pltpu.async_copy(
                            table_hbm.at[idx_v.at[b, pl.ds(j * SUB, SUB)]],
                            rows_v.at[b, pl.ds(j * SUB, SUB)],
                            gsem[b],
                        )
                    )
                for cp in cps:
                    cp.wait()
                pltpu.async_copy(
                    rows_v.at[b], out_hbm.at[pl.ds(off, CHUNK)], ssem[b]
                )
            return carry

        lax.fori_loop(0, nchunk // NBUF, body, 0)

        for b in range(NBUF):
            off = base + (nchunk - NBUF + b) * CHUNK
            pltpu.make_async_copy(
                rows_v.at[b], out_hbm.at[pl.ds(off, CHUNK)], ssem[b]
            ).wait()

    return k(table, ids)


def kernel(input_ids, table):
    batch, seq = input_ids.shape
    ids = input_ids.reshape(-1)
    out = _embed_gather(table, ids)
    return out.reshape(batch, seq, EMBED_DIM)


# speedup vs baseline: 1.8531x; 1.1078x over previous
import functools

import jax
import jax.numpy as jnp
from jax import lax
from jax.experimental import pallas as pl
from jax.experimental.pallas import tpu as pltpu
from jax.experimental.pallas import tpu_sc as plsc

EMBED_DIM = 128
NUM_CORES = 2
NUM_SUBCORES = 16
NW = NUM_CORES * NUM_SUBCORES

CHUNK = 128
NBUF = 4


def _embed_gather(table, ids):
    B = ids.shape[0]
    rows_per_w = B // NW
    nchunk = rows_per_w // CHUNK

    mesh = plsc.VectorSubcoreMesh(core_axis_name="c", subcore_axis_name="s")

    @functools.partial(
        pl.kernel,
        mesh=mesh,
        out_type=jax.ShapeDtypeStruct((B, EMBED_DIM), jnp.float32),
        scratch_types=[
            pltpu.VMEM((NBUF, CHUNK), jnp.int32),
            pltpu.VMEM((NBUF, CHUNK, EMBED_DIM), jnp.float32),
        ]
        + [pltpu.SemaphoreType.DMA] * (3 * NBUF),
    )
    def k(table_hbm, ids_hbm, out_hbm, idx_v, rows_v, *sems):
        gsem = list(sems[0:NBUF])
        ssem = list(sems[NBUF : 2 * NBUF])
        isem = list(sems[2 * NBUF : 3 * NBUF])
        wid = lax.axis_index("s") * NUM_CORES + lax.axis_index("c")
        base = wid * rows_per_w

        def idx_load(c, slot):
            pltpu.async_copy(
                ids_hbm.at[pl.ds(base + c * CHUNK, CHUNK)],
                idx_v.at[slot],
                isem[slot],
            )

        def idx_wait(c, slot):
            pltpu.make_async_copy(
                ids_hbm.at[pl.ds(base + c * CHUNK, CHUNK)],
                idx_v.at[slot],
                isem[slot],
            ).wait()

        def gather_start(slot):
            pltpu.async_copy(
                table_hbm.at[idx_v.at[slot]], rows_v.at[slot], gsem[slot]
            )

        def gather_wait(slot):
            pltpu.make_async_copy(
                table_hbm.at[idx_v.at[slot]], rows_v.at[slot], gsem[slot]
            ).wait()

        def store_start(c, slot):
            pltpu.async_copy(
                rows_v.at[slot],
                out_hbm.at[pl.ds(base + c * CHUNK, CHUNK)],
                ssem[slot],
            )

        def store_wait(c, slot):
            pltpu.make_async_copy(
                rows_v.at[slot],
                out_hbm.at[pl.ds(base + c * CHUNK, CHUNK)],
                ssem[slot],
            ).wait()

        idx_load(0, 0)
        idx_load(1, 1)
        for c in range(NBUF):
            idx_wait(c, c)
            gather_start(c)
            if c >= 1:
                gather_wait(c - 1)
                store_start(c - 1, c - 1)
            idx_load(c + 2, (c + 2) % NBUF)

        def body(g, carry):
            for b in range(NBUF):
                c = NBUF * g + b
                store_wait(c - NBUF, b)
                idx_wait(c, b)
                gather_start(b)
                prev = (b - 1) % NBUF
                gather_wait(prev)
                store_start(c - 1, prev)

                @pl.when(c + 2 < nchunk)
                def _(c=c, b=b):
                    idx_load(c + 2, (b + 2) % NBUF)

            return carry

        lax.fori_loop(1, nchunk // NBUF, body, 0)

        last = nchunk - 1
        gather_wait(last % NBUF)
        store_start(last, last % NBUF)
        for c in range(nchunk - NBUF, nchunk):
            store_wait(c, c % NBUF)

    return k(table, ids)


def kernel(input_ids, table):
    batch, seq = input_ids.shape
    ids = input_ids.reshape(-1)
    out = _embed_gather(table, ids)
    return out.reshape(batch, seq, EMBED_DIM)
